# baseline (speedup 1.0000x reference)
import functools

import jax
import jax.numpy as jnp
from jax import lax
from jax.experimental import pallas as pl
from jax.experimental.pallas import tpu as pltpu
from jax.experimental.pallas import tpu_sc as plsc

_NC = 2
_NS = 16
_L = 16
_NW = _NC * _NS
_CHUNK = 128
_CW = 65536
_W2 = 458752
_SCCH = 1024


def _dense_scores_tc(tableT, wcb, w2):
    D, V = tableT.shape
    nskip = w2 // _CW
    assert w2 % _CW == 0

    def body(t_ref, w_ref, s_ref):
        x = t_ref[...]
        w = w_ref[:, 0:1]
        s_ref[...] = jnp.sum(x * w, axis=0)

    return pl.pallas_call(
        body,
        grid=(pl.cdiv(V - w2, _CW),),
        compiler_params=pltpu.CompilerParams(
            dimension_semantics=("arbitrary",)),
        in_specs=[
            pl.BlockSpec((D, _CW), lambda i: (0, i + nskip)),
            pl.BlockSpec((D, 128), lambda i: (0, 0)),
        ],
        out_specs=pl.BlockSpec((_CW,), lambda i: (i,)),
        out_shape=jax.ShapeDtypeStruct((V - w2,), jnp.float32),
    )(tableT, wcb)


@functools.lru_cache(maxsize=None)
def _build_scan_sc(D, V, w2):
    wpw = w2 // _NW
    nck = wpw // _SCCH
    assert w2 % (_NW * _SCCH) == 0

    mesh = plsc.VectorSubcoreMesh(core_axis_name="c", subcore_axis_name="s")

    @functools.partial(
        pl.kernel,
        mesh=mesh,
        compiler_params=pltpu.CompilerParams(
            needs_layout_passes=False, use_tc_tiling_on_sc=True),
        out_type=jax.ShapeDtypeStruct((w2,), jnp.float32),
        scratch_types=[
            pltpu.VMEM((2, D, _SCCH), jnp.float32),
            pltpu.VMEM(((D + 1) * _L,), jnp.float32),
            pltpu.VMEM((wpw,), jnp.float32),
            pltpu.SemaphoreType.DMA,
        ],
    )
    def scan(aux_hbm, tableT_hbm, s_hbm, stage_v, aux_v, res_v, sem):
        wid = lax.axis_index("s") * _NC + lax.axis_index("c")
        base = wid * wpw
        pltpu.sync_copy(aux_hbm, aux_v)
        lanes16 = lax.iota(jnp.int32, _L)

        def fetch(c, buf):
            return pltpu.async_copy(
                tableT_hbm.at[pl.ds(0, D), pl.ds(base + c * _SCCH, _SCCH)],
                stage_v.at[buf], sem)

        wregs = [aux_v[pl.ds(d * _L, _L)] for d in range(D)]

        fetch(0, 0)
        for c in range(nck):
            if c + 1 < nck:
                fetch(c + 1, (c + 1) % 2)
            pltpu.make_async_copy(
                tableT_hbm.at[pl.ds(0, D), pl.ds(0, _SCCH)],
                stage_v.at[c % 2], sem).wait()

            def sub_body(sub, carry, _c=c):
                cols = sub * _L + lanes16
                acc = jnp.zeros((_L,), jnp.float32)
                for d in range(D):
                    vals = plsc.load_gather(
                        stage_v,
                        [jnp.full((_L,), _c % 2, jnp.int32),
                         jnp.full((_L,), d, jnp.int32), cols])
                    acc = acc + vals * wregs[d]
                res_v[pl.ds(_c * _SCCH + sub * _L, _L)] = acc
                return carry

            lax.fori_loop(0, _SCCH // _L, sub_body, 0)
        pltpu.sync_copy(res_v, s_hbm.at[pl.ds(base, wpw)])

    return scan


@functools.lru_cache(maxsize=None)
def _build_pick(B, w2):
    bpw = B // _NW
    nch = bpw // _CHUNK
    assert B % (_NW * _CHUNK) == 0

    mesh = plsc.VectorSubcoreMesh(core_axis_name="c", subcore_axis_name="s")

    @functools.partial(
        pl.kernel,
        mesh=mesh,
        compiler_params=pltpu.CompilerParams(
            needs_layout_passes=False, use_tc_tiling_on_sc=False),
        out_type=jax.ShapeDtypeStruct((B,), jnp.float32),
        scratch_types=[
            pltpu.VMEM((nch, _CHUNK), jnp.int32),
            pltpu.VMEM((nch, _CHUNK), jnp.int32),
            pltpu.VMEM((nch, _CHUNK), jnp.int32),
            pltpu.VMEM((bpw,), jnp.float32),
            pltpu.VMEM((bpw,), jnp.float32),
            pltpu.VMEM((_L,), jnp.float32),
            pltpu.SemaphoreType.DMA,
        ],
    )
    def pick(idx_hbm, ilo_hbm, ihi_hbm, bias_hbm, slo_hbm, shi_hbm, out_hbm,
             idx_v, ilo_v, ihi_v, vlo_v, vhi_v, bias_v, sem):
        wid = lax.axis_index("s") * _NC + lax.axis_index("c")
        base = wid * bpw
        pltpu.sync_copy(idx_hbm.at[wid], idx_v)
        pltpu.sync_copy(ilo_hbm.at[wid], ilo_v)
        pltpu.sync_copy(ihi_hbm.at[wid], ihi_v)
        pltpu.sync_copy(bias_hbm, bias_v)
        copies = []
        for j in range(nch):
            copies.append(pltpu.async_copy(
                slo_hbm.at[ilo_v.at[j]],
                vlo_v.at[pl.ds(j * _CHUNK, _CHUNK)], sem))
            copies.append(pltpu.async_copy(
                shi_hbm.at[ihi_v.at[j]],
                vhi_v.at[pl.ds(j * _CHUNK, _CHUNK)], sem))
        for h in copies:
            h.wait()
        bias = bias_v[...]
        for j in range(nch):
            for k in range(_CHUNK // _L):
                o = j * _CHUNK + k * _L
                v = idx_v[j, pl.ds(k * _L, _L)]
                sel = jnp.where(v < jnp.int32(w2),
                                vlo_v[pl.ds(o, _L)], vhi_v[pl.ds(o, _L)])
                vlo_v[pl.ds(o, _L)] = sel + bias
        pltpu.sync_copy(vlo_v, out_hbm.at[pl.ds(base, bpw)])

    return pick


def kernel(x_movie, x_user, movie_table, fc_w, fc_b):
    B = x_movie.shape[0]
    V, D = movie_table.shape
    wc = fc_w[0, :D] + fc_w[0, D:]
    wcb = jnp.broadcast_to(wc[:, None], (D, 128))
    aux = jnp.broadcast_to(
        jnp.concatenate([wc, fc_b])[:, None], (D + 1, _L)
    ).astype(jnp.float32).reshape(-1)
    bias = jnp.broadcast_to(fc_b, (_L,)).astype(jnp.float32)
    idx32 = x_movie.astype(jnp.int32)
    idx = idx32.reshape(_NW, B // (_NW * _CHUNK), _CHUNK)
    ilo = jnp.minimum(idx, _W2 - 1)
    ihi = jnp.maximum(idx - _W2, 0)
    tableT = movie_table.T
    s_lo = _build_scan_sc(D, V, _W2)(aux, tableT)
    s_hi = _dense_scores_tc(tableT, wcb, _W2)
    out = _build_pick(B, _W2)(idx, ilo, ihi, bias, s_lo, s_hi)
    return out.reshape(B, 1)


# speedup vs baseline: 1.4654x; 1.4654x over previous
import functools

import jax
import jax.numpy as jnp
from jax import lax
from jax.experimental import pallas as pl
from jax.experimental.pallas import tpu as pltpu
from jax.experimental.pallas import tpu_sc as plsc

_NC = 2
_NS = 16
_L = 16
_NW = _NC * _NS
_CHUNK = 128
_CW = 65536
_W2 = 458752
_SCCH = 1024


def _dense_scores_tc(tableT, wcb, w2):
    D, V = tableT.shape
    nskip = w2 // _CW
    assert w2 % _CW == 0

    def body(t_ref, w_ref, s_ref):
        x = t_ref[...]
        w = w_ref[:, 0:1]
        s_ref[...] = jnp.sum(x * w, axis=0)

    return pl.pallas_call(
        body,
        grid=(pl.cdiv(V - w2, _CW),),
        compiler_params=pltpu.CompilerParams(
            dimension_semantics=("arbitrary",)),
        in_specs=[
            pl.BlockSpec((D, _CW), lambda i: (0, i + nskip)),
            pl.BlockSpec((D, 128), lambda i: (0, 0)),
        ],
        out_specs=pl.BlockSpec((_CW,), lambda i: (i,)),
        out_shape=jax.ShapeDtypeStruct((V - w2,), jnp.float32),
    )(tableT, wcb)


@functools.lru_cache(maxsize=None)
def _build_scan_sc(D, V, w2):
    wpw = w2 // _NW
    nck = wpw // _SCCH
    assert w2 % (_NW * _SCCH) == 0

    mesh = plsc.VectorSubcoreMesh(core_axis_name="c", subcore_axis_name="s")

    @functools.partial(
        pl.kernel,
        mesh=mesh,
        compiler_params=pltpu.CompilerParams(
            needs_layout_passes=False, use_tc_tiling_on_sc=True),
        out_type=jax.ShapeDtypeStruct((w2,), jnp.float32),
        scratch_types=[
            pltpu.VMEM((2, D, _SCCH), jnp.float32),
            pltpu.VMEM(((D + 1) * _L,), jnp.float32),
            pltpu.VMEM((wpw,), jnp.float32),
            pltpu.SemaphoreType.DMA,
        ],
    )
    def scan(aux_hbm, tableT_hbm, s_hbm, stage_v, aux_v, res_v, sem):
        wid = lax.axis_index("s") * _NC + lax.axis_index("c")
        base = wid * wpw
        pltpu.sync_copy(aux_hbm, aux_v)
        lanes16 = lax.iota(jnp.int32, _L)

        def fetch(c, buf):
            return pltpu.async_copy(
                tableT_hbm.at[pl.ds(0, D), pl.ds(base + c * _SCCH, _SCCH)],
                stage_v.at[buf], sem)

        wregs = [aux_v[pl.ds(d * _L, _L)] for d in range(D)]

        fetch(0, 0)
        for c in range(nck):
            if c + 1 < nck:
                fetch(c + 1, (c + 1) % 2)
            pltpu.make_async_copy(
                tableT_hbm.at[pl.ds(0, D), pl.ds(0, _SCCH)],
                stage_v.at[c % 2], sem).wait()

            def sub_body(sub, carry, _c=c):
                cols = sub * _L + lanes16
                acc = jnp.zeros((_L,), jnp.float32)
                for d in range(D):
                    vals = plsc.load_gather(
                        stage_v,
                        [jnp.full((_L,), _c % 2, jnp.int32),
                         jnp.full((_L,), d, jnp.int32), cols])
                    acc = acc + vals * wregs[d]
                res_v[pl.ds(_c * _SCCH + sub * _L, _L)] = acc
                return carry

            lax.fori_loop(0, _SCCH // _L, sub_body, 0)
        pltpu.sync_copy(res_v, s_hbm.at[pl.ds(base, wpw)])

    return scan


@functools.lru_cache(maxsize=None)
def _build_pick(B, w2):
    bpw = B // _NW
    nch = bpw // _CHUNK
    assert B % (_NW * _CHUNK) == 0

    mesh = plsc.VectorSubcoreMesh(core_axis_name="c", subcore_axis_name="s")

    @functools.partial(
        pl.kernel,
        mesh=mesh,
        compiler_params=pltpu.CompilerParams(
            needs_layout_passes=False, use_tc_tiling_on_sc=False),
        out_type=jax.ShapeDtypeStruct((B,), jnp.float32),
        scratch_types=[
            pltpu.VMEM((nch, _CHUNK), jnp.int32),
            pltpu.VMEM((bpw,), jnp.float32),
            pltpu.VMEM((_L,), jnp.float32),
            pltpu.SemaphoreType.DMA,
        ],
    )
    def pick(idx_hbm, bias_hbm, s_hbm, out_hbm, idx_v, val_v, bias_v, sem):
        wid = lax.axis_index("s") * _NC + lax.axis_index("c")
        base = wid * bpw
        pltpu.sync_copy(idx_hbm.at[wid], idx_v)
        pltpu.sync_copy(bias_hbm, bias_v)
        copies = [
            pltpu.async_copy(
                s_hbm.at[idx_v.at[j]],
                val_v.at[pl.ds(j * _CHUNK, _CHUNK)], sem)
            for j in range(nch)
        ]
        for h in copies:
            h.wait()
        bias = bias_v[...]
        for j in range(bpw // _L):
            o = j * _L
            val_v[pl.ds(o, _L)] = val_v[pl.ds(o, _L)] + bias
        pltpu.sync_copy(val_v, out_hbm.at[pl.ds(base, bpw)])

    return pick


def kernel(x_movie, x_user, movie_table, fc_w, fc_b):
    B = x_movie.shape[0]
    V, D = movie_table.shape
    wc = fc_w[0, :D] + fc_w[0, D:]
    wcb = jnp.broadcast_to(wc[:, None], (D, 128))
    aux = jnp.broadcast_to(
        jnp.concatenate([wc, fc_b])[:, None], (D + 1, _L)
    ).astype(jnp.float32).reshape(-1)
    bias = jnp.broadcast_to(fc_b, (_L,)).astype(jnp.float32)
    idx = x_movie.astype(jnp.int32).reshape(_NW, B // (_NW * _CHUNK), _CHUNK)
    tableT = movie_table.T
    s_lo = _build_scan_sc(D, V, _W2)(aux, tableT)
    s_hi = _dense_scores_tc(tableT, wcb, _W2)
    s = jnp.concatenate([s_lo, s_hi])
    out = _build_pick(B, _W2)(idx, bias, s)
    return out.reshape(B, 1)


# speedup vs baseline: 1.4688x; 1.0023x over previous
import functools

import jax
import jax.numpy as jnp
from jax import lax
from jax.experimental import pallas as pl
from jax.experimental.pallas import tpu as pltpu
from jax.experimental.pallas import tpu_sc as plsc

_NC = 2
_NS = 16
_L = 16
_NW = _NC * _NS
_CHUNK = 128
_CW = 65536
_W2 = 458752
_SCCH = 1024


def _dense_scores_tc(tableT, wcb, w2):
    D, V = tableT.shape
    nskip = w2 // _CW
    assert w2 % _CW == 0

    def body(t_ref, w_ref, s_ref):
        x = t_ref[...]
        w = w_ref[:, 0:1]
        s_ref[...] = jnp.sum(x * w, axis=0)

    return pl.pallas_call(
        body,
        grid=(pl.cdiv(V - w2, _CW),),
        compiler_params=pltpu.CompilerParams(
            dimension_semantics=("arbitrary",), skip_device_barrier=True),
        in_specs=[
            pl.BlockSpec((D, _CW), lambda i: (0, i + nskip)),
            pl.BlockSpec((D, 128), lambda i: (0, 0)),
        ],
        out_specs=pl.BlockSpec((_CW,), lambda i: (i,)),
        out_shape=jax.ShapeDtypeStruct((V - w2,), jnp.float32),
    )(tableT, wcb)


@functools.lru_cache(maxsize=None)
def _build_scan_sc(D, V, w2):
    wpw = w2 // _NW
    nck = wpw // _SCCH
    assert w2 % (_NW * _SCCH) == 0

    mesh = plsc.VectorSubcoreMesh(core_axis_name="c", subcore_axis_name="s")

    @functools.partial(
        pl.kernel,
        mesh=mesh,
        compiler_params=pltpu.CompilerParams(
            needs_layout_passes=False, use_tc_tiling_on_sc=True),
        out_type=jax.ShapeDtypeStruct((w2,), jnp.float32),
        scratch_types=[
            pltpu.VMEM((2, D, _SCCH), jnp.float32),
            pltpu.VMEM(((D + 1) * _L,), jnp.float32),
            pltpu.VMEM((wpw,), jnp.float32),
            pltpu.SemaphoreType.DMA,
        ],
    )
    def scan(aux_hbm, tableT_hbm, s_hbm, stage_v, aux_v, res_v, sem):
        wid = lax.axis_index("s") * _NC + lax.axis_index("c")
        base = wid * wpw
        pltpu.sync_copy(aux_hbm, aux_v)
        lanes16 = lax.iota(jnp.int32, _L)

        def fetch(c, buf):
            return pltpu.async_copy(
                tableT_hbm.at[pl.ds(0, D), pl.ds(base + c * _SCCH, _SCCH)],
                stage_v.at[buf], sem)

        wregs = [aux_v[pl.ds(d * _L, _L)] for d in range(D)]

        fetch(0, 0)
        for c in range(nck):
            if c + 1 < nck:
                fetch(c + 1, (c + 1) % 2)
            pltpu.make_async_copy(
                tableT_hbm.at[pl.ds(0, D), pl.ds(0, _SCCH)],
                stage_v.at[c % 2], sem).wait()

            def sub_body(sub, carry, _c=c):
                cols = sub * _L + lanes16
                acc = jnp.zeros((_L,), jnp.float32)
                for d in range(D):
                    vals = plsc.load_gather(
                        stage_v,
                        [jnp.full((_L,), _c % 2, jnp.int32),
                         jnp.full((_L,), d, jnp.int32), cols])
                    acc = acc + vals * wregs[d]
                res_v[pl.ds(_c * _SCCH + sub * _L, _L)] = acc
                return carry

            lax.fori_loop(0, _SCCH // _L, sub_body, 0)
        pltpu.sync_copy(res_v, s_hbm.at[pl.ds(base, wpw)])

    return scan


@functools.lru_cache(maxsize=None)
def _build_pick(B, w2):
    bpw = B // _NW
    nch = bpw // _CHUNK
    assert B % (_NW * _CHUNK) == 0

    mesh = plsc.VectorSubcoreMesh(core_axis_name="c", subcore_axis_name="s")

    @functools.partial(
        pl.kernel,
        mesh=mesh,
        compiler_params=pltpu.CompilerParams(
            needs_layout_passes=False, use_tc_tiling_on_sc=False),
        out_type=jax.ShapeDtypeStruct((B,), jnp.float32),
        scratch_types=[
            pltpu.VMEM((nch, _CHUNK), jnp.int32),
            pltpu.VMEM((bpw,), jnp.float32),
            pltpu.VMEM((_L,), jnp.float32),
            pltpu.SemaphoreType.DMA,
        ],
    )
    def pick(idx_hbm, bias_hbm, s_hbm, out_hbm, idx_v, val_v, bias_v, sem):
        wid = lax.axis_index("s") * _NC + lax.axis_index("c")
        base = wid * bpw
        pltpu.sync_copy(idx_hbm.at[wid], idx_v)
        pltpu.sync_copy(bias_hbm, bias_v)
        copies = [
            pltpu.async_copy(
                s_hbm.at[idx_v.at[j]],
                val_v.at[pl.ds(j * _CHUNK, _CHUNK)], sem)
            for j in range(nch)
        ]
        for h in copies:
            h.wait()
        bias = bias_v[...]
        for j in range(bpw // _L):
            o = j * _L
            val_v[pl.ds(o, _L)] = val_v[pl.ds(o, _L)] + bias
        pltpu.sync_copy(val_v, out_hbm.at[pl.ds(base, bpw)])

    return pick


def kernel(x_movie, x_user, movie_table, fc_w, fc_b):
    B = x_movie.shape[0]
    V, D = movie_table.shape
    wc = fc_w[0, :D] + fc_w[0, D:]
    wcb = jnp.broadcast_to(wc[:, None], (D, 128))
    aux = jnp.broadcast_to(
        jnp.concatenate([wc, fc_b])[:, None], (D + 1, _L)
    ).astype(jnp.float32).reshape(-1)
    bias = jnp.broadcast_to(fc_b, (_L,)).astype(jnp.float32)
    idx = x_movie.astype(jnp.int32).reshape(_NW, B // (_NW * _CHUNK), _CHUNK)
    tableT = movie_table.T
    s_lo = _build_scan_sc(D, V, _W2)(aux, tableT)
    s_hi = _dense_scores_tc(tableT, wcb, _W2)
    s = jnp.concatenate([s_lo, s_hi])
    out = _build_pick(B, _W2)(idx, bias, s)
    return out.reshape(B, 1)


# speedup vs baseline: 1.8899x; 1.2867x over previous
import functools

import jax
import jax.numpy as jnp
from jax import lax
from jax.experimental import pallas as pl
from jax.experimental.pallas import tpu as pltpu
from jax.experimental.pallas import tpu_sc as plsc

_NC = 2
_NS = 16
_L = 16
_NW = _NC * _NS
_CHUNK = 128
_CW = 65536


def _dense_scores(tableT, wcb):
    D, V = tableT.shape

    def body(t_ref, w_ref, s_ref):
        x = t_ref[...]
        w = w_ref[:, 0:1]
        s_ref[...] = jnp.sum(x * w, axis=0)

    return pl.pallas_call(
        body,
        grid=(pl.cdiv(V, _CW),),
        compiler_params=pltpu.CompilerParams(
            dimension_semantics=("parallel",)),
        in_specs=[
            pl.BlockSpec((D, _CW), lambda i: (0, i)),
            pl.BlockSpec((D, 128), lambda i: (0, 0)),
        ],
        out_specs=pl.BlockSpec((_CW,), lambda i: (i,)),
        out_shape=jax.ShapeDtypeStruct((V,), jnp.float32),
    )(tableT, wcb)


@functools.lru_cache(maxsize=None)
def _build_pick(B, V):
    assert B % (_NW * _CHUNK) == 0
    bpw = B // _NW
    nch = bpw // _CHUNK

    mesh = plsc.VectorSubcoreMesh(core_axis_name="c", subcore_axis_name="s")

    @functools.partial(
        pl.kernel,
        mesh=mesh,
        compiler_params=pltpu.CompilerParams(
            needs_layout_passes=False, use_tc_tiling_on_sc=False),
        out_type=jax.ShapeDtypeStruct((B,), jnp.float32),
        scratch_types=[
            pltpu.VMEM((nch, _CHUNK), jnp.int32),
            pltpu.VMEM((bpw,), jnp.float32),
            pltpu.VMEM((_L,), jnp.float32),
            pltpu.SemaphoreType.DMA,
        ],
    )
    def pick(idx_hbm, bias_hbm, s_hbm, out_hbm, idx_v, val_v, bias_v, sem):
        wid = lax.axis_index("s") * _NC + lax.axis_index("c")
        base = wid * bpw
        pltpu.sync_copy(idx_hbm.at[wid], idx_v)
        pltpu.sync_copy(bias_hbm, bias_v)
        copies = [
            pltpu.async_copy(
                s_hbm.at[idx_v.at[j]],
                val_v.at[pl.ds(j * _CHUNK, _CHUNK)], sem)
            for j in range(nch)
        ]
        for h in copies:
            h.wait()
        bias = bias_v[...]
        for j in range(bpw // _L):
            o = j * _L
            val_v[pl.ds(o, _L)] = val_v[pl.ds(o, _L)] + bias
        pltpu.sync_copy(val_v, out_hbm.at[pl.ds(base, bpw)])

    return pick


def kernel(x_movie, x_user, movie_table, fc_w, fc_b):
    B = x_movie.shape[0]
    V, D = movie_table.shape
    wc = fc_w[0, :D] + fc_w[0, D:]
    wcb = jnp.broadcast_to(wc[:, None], (D, 128))
    bias = jnp.broadcast_to(fc_b, (_L,)).astype(jnp.float32)
    idx = x_movie.astype(jnp.int32).reshape(_NW, B // (_NW * _CHUNK), _CHUNK)
    s = _dense_scores(movie_table.T, wcb)
    out = _build_pick(B, V)(idx, bias, s)
    return out.reshape(B, 1)
